# baseline (speedup 1.0000x reference)
import functools

import jax
import jax.numpy as jnp
from jax import lax
from jax.experimental import pallas as pl
from jax.experimental.pallas import tpu as pltpu
from jax.experimental.pallas import tpu_sc as plsc

N = 10000
E = 320000
DH = 128
NC = 2
NS = 16
CHUNK = 128
NCHUNK = 160
EPAD = NS * NCHUNK * CHUNK
ACC_ROWS = 10240
ROWS_PER_TILE = ACC_ROWS // NS

_mesh = plsc.VectorSubcoreMesh(
    core_axis_name="c", subcore_axis_name="s", num_cores=NC, num_subcores=NS)

f32 = jnp.float32


def _fill_buf(buf, rows, cols, value):
    vec = jnp.full((16,), value, dtype=f32)

    def body(r, _):
        for k in range(cols // 16):
            buf[r, pl.ds(k * 16, 16)] = vec
        return 0

    lax.fori_loop(0, rows, body, 0)


def _deg_body(dst_hbm, p0_hbm, p1_hbm, dst_v, buf, acc_sh, sem):
    cid = lax.axis_index("c")
    sid = lax.axis_index("s")

    pltpu.sync_copy(dst_hbm.at[sid], dst_v)

    _fill_buf(buf, CHUNK, DH, 0.0)
    for k in range(ROWS_PER_TILE // CHUNK):
        pltpu.sync_copy(buf, acc_sh.at[pl.ds(sid * ROWS_PER_TILE + k * CHUNK, CHUNK)])
    plsc.subcore_barrier()

    _fill_buf(buf, CHUNK, DH, 1.0)
    lo = cid * PH0
    hi = PH0 + cid * (NCHUNK - PH0)

    def group(gi, _):
        base = lo + gi * 8
        for t in range(8):
            j = base + t

            @pl.when(j < hi)
            def _():
                pltpu.async_copy(buf, acc_sh.at[dst_v.at[j]], sem, add=True)

        for t in range(8):
            j = base + t

            @pl.when(j < hi)
            def _():
                pltpu.make_async_copy(buf, acc_sh.at[dst_v.at[j]], sem).wait()

        return 0

    lax.fori_loop(0, (PH0 + 7) // 8, group, 0)
    plsc.subcore_barrier()

    my_rows = pl.ds(sid * ROWS_PER_TILE, ROWS_PER_TILE)

    @pl.when(cid == 0)
    def _():
        pltpu.sync_copy(acc_sh.at[my_rows], p0_hbm.at[my_rows])

    @pl.when(cid == 1)
    def _():
        pltpu.sync_copy(acc_sh.at[my_rows], p1_hbm.at[my_rows])


_deg_kernel = functools.partial(
    pl.kernel,
    out_type=[jax.ShapeDtypeStruct((ACC_ROWS, DH), f32)] * 2,
    mesh=_mesh,
    scratch_types=[
        pltpu.VMEM((NCHUNK, CHUNK), jnp.int32),
        pltpu.VMEM((CHUNK, DH), f32),
        pltpu.VMEM_SHARED((ACC_ROWS, DH), f32),
        pltpu.SemaphoreType.DMA,
    ],
)(_deg_body)


PH0 = 80


def _agg_half(g_hbm, o_hbm, pk_hbm, pk_v, sidx, didx, bufA, bufB, acc_sh,
              semG, semSA, semSB, sid):
    def unpack(r, w):
        for k in range(CHUNK // 16):
            p = pk_v[r, pl.ds(k * 16, 16)]
            sidx[w, pl.ds(k * 16, 16)] = jnp.bitwise_and(p, 0x3FFF)
            didx[w, pl.ds(k * 16, 16)] = lax.shift_right_logical(p, 14)

    def drainA():
        pltpu.make_async_copy(bufA, acc_sh.at[didx.at[0]], semSA).wait()

    def drainB():
        pltpu.make_async_copy(bufB, acc_sh.at[didx.at[1]], semSB).wait()

    for off in (0, PH0):
        pltpu.sync_copy(pk_hbm.at[sid].at[pl.ds(off, PH0)], pk_v)

        def body(i, _):
            j = 2 * i

            @pl.when(i > 0)
            def _():
                drainA()

            unpack(j, 0)
            pltpu.async_copy(g_hbm.at[sidx.at[0]], bufA, semG).wait()
            pltpu.async_copy(bufA, acc_sh.at[didx.at[0]], semSA, add=True)

            @pl.when(i > 0)
            def _():
                drainB()

            unpack(j + 1, 1)
            pltpu.async_copy(g_hbm.at[sidx.at[1]], bufB, semG).wait()
            pltpu.async_copy(bufB, acc_sh.at[didx.at[1]], semSB, add=True)

            return 0

        lax.fori_loop(0, PH0 // 2, body, 0)
        drainA()
        drainB()

    plsc.subcore_barrier()
    my_rows = pl.ds(sid * ROWS_PER_TILE, ROWS_PER_TILE)
    pltpu.sync_copy(acc_sh.at[my_rows], o_hbm.at[my_rows])


def _agg_body(g0_hbm, g1_hbm, pk_hbm, o0_hbm, o1_hbm,
              pk_v, sidx, didx, bufA, bufB, acc_sh, semG, semSA, semSB):
    cid = lax.axis_index("c")
    sid = lax.axis_index("s")

    _fill_buf(bufA, CHUNK, DH, 0.0)
    for k in range(ROWS_PER_TILE // CHUNK):
        pltpu.sync_copy(bufA, acc_sh.at[pl.ds(sid * ROWS_PER_TILE + k * CHUNK, CHUNK)])
    plsc.subcore_barrier()

    @pl.when(cid == 0)
    def _():
        _agg_half(g0_hbm, o0_hbm, pk_hbm, pk_v, sidx, didx, bufA, bufB,
                  acc_sh, semG, semSA, semSB, sid)

    @pl.when(cid == 1)
    def _():
        _agg_half(g1_hbm, o1_hbm, pk_hbm, pk_v, sidx, didx, bufA, bufB,
                  acc_sh, semG, semSA, semSB, sid)


_agg_kernel = functools.partial(
    pl.kernel,
    out_type=[jax.ShapeDtypeStruct((ACC_ROWS, DH), f32)] * 2,
    mesh=_mesh,
    scratch_types=[
        pltpu.VMEM((PH0, CHUNK), jnp.int32),
        pltpu.VMEM((2, CHUNK), jnp.int32),
        pltpu.VMEM((2, CHUNK), jnp.int32),
        pltpu.VMEM((CHUNK, DH), f32),
        pltpu.VMEM((CHUNK, DH), f32),
        pltpu.VMEM_SHARED((ACC_ROWS, DH), f32),
        pltpu.SemaphoreType.DMA,
        pltpu.SemaphoreType.DMA,
        pltpu.SemaphoreType.DMA,
    ],
)(_agg_body)


RB = 2000
NRB = N // RB


def _dis(p0_ref, p1_ref):
    deg = p0_ref[:, 0:1] + p1_ref[:, 0:1] + 1.0
    return lax.rsqrt(deg)


def _dot(a, b):
    return jnp.dot(a, b, precision=lax.Precision.HIGHEST,
                   preferred_element_type=f32)


def _tc_in_body(x_ref, w_ref, p0_ref, p1_ref, g0_ref, g1_ref):
    dis = _dis(p0_ref, p1_ref)
    z = _dot(x_ref[...], w_ref[...])
    g0_ref[...] = dis * z[:, 0:DH]
    g1_ref[...] = dis * z[:, DH:2 * DH]


def _tc_mid_body(s0_ref, s1_ref, g0_ref, g1_ref, p0_ref, p1_ref, b_ref, w_ref,
                 o0_ref, o1_ref):
    dis = _dis(p0_ref, p1_ref)
    h0 = jnp.maximum(dis * (s0_ref[...] + g0_ref[...]) + b_ref[0:1, 0:DH], 0.0)
    h1 = jnp.maximum(dis * (s1_ref[...] + g1_ref[...]) + b_ref[0:1, DH:2 * DH], 0.0)
    z = _dot(jnp.concatenate([h0, h1], axis=1), w_ref[...])
    o0_ref[...] = dis * z[:, 0:DH]
    o1_ref[...] = dis * z[:, DH:2 * DH]


def _tc_out_body(s0_ref, s1_ref, g0_ref, g1_ref, p0_ref, p1_ref, b_ref, w_ref,
                 bout_ref, o_ref):
    dis = _dis(p0_ref, p1_ref)
    h0 = dis * (s0_ref[...] + g0_ref[...]) + b_ref[0:1, 0:DH]
    h1 = dis * (s1_ref[...] + g1_ref[...]) + b_ref[0:1, DH:2 * DH]
    o_ref[...] = _dot(jnp.concatenate([h0, h1], axis=1), w_ref[...]) + bout_ref[0:1, :]


def _row_blk(cols):
    return pl.BlockSpec((RB, cols), lambda i: (i, 0))


def _full_blk(r, c):
    return pl.BlockSpec((r, c), lambda i: (0, 0))


_tc_in = pl.pallas_call(
    _tc_in_body,
    grid=(NRB,),
    in_specs=[_row_blk(DH), _full_blk(DH, 2 * DH), _row_blk(DH), _row_blk(DH)],
    out_specs=[_row_blk(DH), _row_blk(DH)],
    out_shape=[jax.ShapeDtypeStruct((N, DH), f32)] * 2,
)

_tc_mid = pl.pallas_call(
    _tc_mid_body,
    grid=(NRB,),
    in_specs=[_row_blk(DH), _row_blk(DH), _row_blk(DH), _row_blk(DH),
              _row_blk(DH), _row_blk(DH), _full_blk(1, 2 * DH),
              _full_blk(2 * DH, 2 * DH)],
    out_specs=[_row_blk(DH), _row_blk(DH)],
    out_shape=[jax.ShapeDtypeStruct((N, DH), f32)] * 2,
)

_tc_out = pl.pallas_call(
    _tc_out_body,
    grid=(NRB,),
    in_specs=[_row_blk(DH), _row_blk(DH), _row_blk(DH), _row_blk(DH),
              _row_blk(DH), _row_blk(DH), _full_blk(1, 2 * DH),
              _full_blk(2 * DH, 2 * DH), _full_blk(1, 2 * DH)],
    out_specs=pl.BlockSpec((RB, 2 * DH), lambda i: (i, 0)),
    out_shape=jax.ShapeDtypeStruct((N, 2 * DH), f32),
)


def kernel(x, edge_index, W1, b1, W2, b2, W3, b3, Wout, bout):
    src = edge_index[0].astype(jnp.int32)
    dst = edge_index[1].astype(jnp.int32)
    pad = EPAD - E
    src_p = jnp.concatenate([src, jnp.zeros((pad,), jnp.int32)])
    dst_p = jnp.concatenate([dst, jnp.full((pad,), N, jnp.int32)])
    dst_t = dst_p.reshape(NS, NCHUNK, CHUNK)
    pk_t = (jnp.left_shift(dst_p, 14) | src_p).reshape(NS, NCHUNK, CHUNK)

    p0, p1 = _deg_kernel(dst_t)

    b1r = b1.reshape(1, 2 * DH)
    b2r = b2.reshape(1, 2 * DH)
    b3r = b3.reshape(1, 2 * DH)
    boutr = bout.reshape(1, 2 * DH)

    g0, g1 = _tc_in(x, W1, p0, p1)
    s0, s1 = _agg_kernel(g0, g1, pk_t)
    g0, g1 = _tc_mid(s0, s1, g0, g1, p0, p1, b1r, W2)
    s0, s1 = _agg_kernel(g0, g1, pk_t)
    g0, g1 = _tc_mid(s0, s1, g0, g1, p0, p1, b2r, W3)
    s0, s1 = _agg_kernel(g0, g1, pk_t)
    return _tc_out(s0, s1, g0, g1, p0, p1, b3r, Wout, boutr)


# speedup vs baseline: 1.4951x; 1.4951x over previous
import functools

import jax
import jax.numpy as jnp
from jax import lax
from jax.experimental import pallas as pl
from jax.experimental.pallas import tpu as pltpu
from jax.experimental.pallas import tpu_sc as plsc

N = 10000
E = 320000
DH = 128
NC = 2
NS = 16
CHUNK = 128
NCHUNK = 157
EPAD = NS * NCHUNK * CHUNK
ACC_ROWS = 10240
ROWS_PER_TILE = ACC_ROWS // NS

_mesh = plsc.VectorSubcoreMesh(
    core_axis_name="c", subcore_axis_name="s", num_cores=NC, num_subcores=NS)

f32 = jnp.float32


def _fill_buf(buf, rows, cols, value):
    vec = jnp.full((16,), value, dtype=f32)

    def body(r, _):
        for k in range(cols // 16):
            buf[r, pl.ds(k * 16, 16)] = vec
        return 0

    lax.fori_loop(0, rows, body, 0)


def _deg_body(dst_hbm, p0_hbm, p1_hbm, dst_v, buf, acc_sh, sem):
    cid = lax.axis_index("c")
    sid = lax.axis_index("s")

    pltpu.sync_copy(dst_hbm.at[sid], dst_v)

    _fill_buf(buf, CHUNK, DH, 0.0)
    for k in range(ROWS_PER_TILE // CHUNK):
        pltpu.sync_copy(buf, acc_sh.at[pl.ds(sid * ROWS_PER_TILE + k * CHUNK, CHUNK)])
    plsc.subcore_barrier()

    _fill_buf(buf, CHUNK, DH, 1.0)
    lo = cid * PH0
    hi = PH0 + cid * (NCHUNK - PH0)

    def group(gi, _):
        base = lo + gi * 8
        for t in range(8):
            j = base + t

            @pl.when(j < hi)
            def _():
                pltpu.async_copy(buf, acc_sh.at[dst_v.at[j]], sem, add=True)

        for t in range(8):
            j = base + t

            @pl.when(j < hi)
            def _():
                pltpu.make_async_copy(buf, acc_sh.at[dst_v.at[j]], sem).wait()

        return 0

    lax.fori_loop(0, (PH0 + 7) // 8, group, 0)
    plsc.subcore_barrier()

    my_rows = pl.ds(sid * ROWS_PER_TILE, ROWS_PER_TILE)

    @pl.when(cid == 0)
    def _():
        pltpu.sync_copy(acc_sh.at[my_rows], p0_hbm.at[my_rows])

    @pl.when(cid == 1)
    def _():
        pltpu.sync_copy(acc_sh.at[my_rows], p1_hbm.at[my_rows])


_deg_kernel = functools.partial(
    pl.kernel,
    out_type=[jax.ShapeDtypeStruct((ACC_ROWS, DH), f32)] * 2,
    mesh=_mesh,
    scratch_types=[
        pltpu.VMEM((NCHUNK, CHUNK), jnp.int32),
        pltpu.VMEM((CHUNK, DH), f32),
        pltpu.VMEM_SHARED((ACC_ROWS, DH), f32),
        pltpu.SemaphoreType.DMA,
    ],
)(_deg_body)


PH0 = 80


def _agg_half(g_hbm, o_hbm, pk_v, sidx, didx, buf, acc_sh, sem, sid):
    def body(j, _):
        for k in range(CHUNK // 16):
            p = pk_v[j, pl.ds(k * 16, 16)]
            sidx[0, pl.ds(k * 16, 16)] = jnp.bitwise_and(p, 0x3FFF)
            didx[0, pl.ds(k * 16, 16)] = lax.shift_right_logical(p, 14)
        pltpu.async_copy(g_hbm.at[sidx.at[0]], buf, sem).wait()
        pltpu.sync_copy(buf, acc_sh.at[didx.at[0]], add=True)
        return 0

    lax.fori_loop(0, NCHUNK, body, 0)
    plsc.subcore_barrier()
    my_rows = pl.ds(sid * ROWS_PER_TILE, ROWS_PER_TILE)
    pltpu.sync_copy(acc_sh.at[my_rows], o_hbm.at[my_rows])


def _agg_body(g0_hbm, g1_hbm, pk_hbm, o0_hbm, o1_hbm,
              pk_v, sidx, didx, buf, acc_sh, sem):
    cid = lax.axis_index("c")
    sid = lax.axis_index("s")

    pltpu.sync_copy(pk_hbm.at[sid], pk_v)

    _fill_buf(buf, CHUNK, DH, 0.0)
    for k in range(ROWS_PER_TILE // CHUNK):
        pltpu.sync_copy(buf, acc_sh.at[pl.ds(sid * ROWS_PER_TILE + k * CHUNK, CHUNK)])
    plsc.subcore_barrier()

    @pl.when(cid == 0)
    def _():
        _agg_half(g0_hbm, o0_hbm, pk_v, sidx, didx, buf, acc_sh, sem, sid)

    @pl.when(cid == 1)
    def _():
        _agg_half(g1_hbm, o1_hbm, pk_v, sidx, didx, buf, acc_sh, sem, sid)


_agg_kernel = functools.partial(
    pl.kernel,
    out_type=[jax.ShapeDtypeStruct((ACC_ROWS, DH), f32)] * 2,
    mesh=_mesh,
    scratch_types=[
        pltpu.VMEM((NCHUNK, CHUNK), jnp.int32),
        pltpu.VMEM((1, CHUNK), jnp.int32),
        pltpu.VMEM((1, CHUNK), jnp.int32),
        pltpu.VMEM((CHUNK, DH), f32),
        pltpu.VMEM_SHARED((ACC_ROWS, DH), f32),
        pltpu.SemaphoreType.DMA,
    ],
)(_agg_body)


RB = 2000
NRB = N // RB


def _dis(p0_ref, p1_ref):
    deg = p0_ref[:, 0:1] + p1_ref[:, 0:1] + 1.0
    return lax.rsqrt(deg)


def _dot(a, b):
    return jnp.dot(a, b, precision=lax.Precision.HIGHEST,
                   preferred_element_type=f32)


def _tc_in_body(x_ref, w_ref, p0_ref, p1_ref, g0_ref, g1_ref):
    dis = _dis(p0_ref, p1_ref)
    z = _dot(x_ref[...], w_ref[...])
    g0_ref[...] = dis * z[:, 0:DH]
    g1_ref[...] = dis * z[:, DH:2 * DH]


def _tc_mid_body(s0_ref, s1_ref, g0_ref, g1_ref, p0_ref, p1_ref, b_ref, w_ref,
                 o0_ref, o1_ref):
    dis = _dis(p0_ref, p1_ref)
    h0 = jnp.maximum(dis * (s0_ref[...] + g0_ref[...]) + b_ref[0:1, 0:DH], 0.0)
    h1 = jnp.maximum(dis * (s1_ref[...] + g1_ref[...]) + b_ref[0:1, DH:2 * DH], 0.0)
    z = _dot(jnp.concatenate([h0, h1], axis=1), w_ref[...])
    o0_ref[...] = dis * z[:, 0:DH]
    o1_ref[...] = dis * z[:, DH:2 * DH]


def _tc_out_body(s0_ref, s1_ref, g0_ref, g1_ref, p0_ref, p1_ref, b_ref, w_ref,
                 bout_ref, o_ref):
    dis = _dis(p0_ref, p1_ref)
    h0 = dis * (s0_ref[...] + g0_ref[...]) + b_ref[0:1, 0:DH]
    h1 = dis * (s1_ref[...] + g1_ref[...]) + b_ref[0:1, DH:2 * DH]
    o_ref[...] = _dot(jnp.concatenate([h0, h1], axis=1), w_ref[...]) + bout_ref[0:1, :]


def _row_blk(cols):
    return pl.BlockSpec((RB, cols), lambda i: (i, 0))


def _full_blk(r, c):
    return pl.BlockSpec((r, c), lambda i: (0, 0))


_tc_in = pl.pallas_call(
    _tc_in_body,
    grid=(NRB,),
    in_specs=[_row_blk(DH), _full_blk(DH, 2 * DH), _row_blk(DH), _row_blk(DH)],
    out_specs=[_row_blk(DH), _row_blk(DH)],
    out_shape=[jax.ShapeDtypeStruct((N, DH), f32)] * 2,
)

_tc_mid = pl.pallas_call(
    _tc_mid_body,
    grid=(NRB,),
    in_specs=[_row_blk(DH), _row_blk(DH), _row_blk(DH), _row_blk(DH),
              _row_blk(DH), _row_blk(DH), _full_blk(1, 2 * DH),
              _full_blk(2 * DH, 2 * DH)],
    out_specs=[_row_blk(DH), _row_blk(DH)],
    out_shape=[jax.ShapeDtypeStruct((N, DH), f32)] * 2,
)

_tc_out = pl.pallas_call(
    _tc_out_body,
    grid=(NRB,),
    in_specs=[_row_blk(DH), _row_blk(DH), _row_blk(DH), _row_blk(DH),
              _row_blk(DH), _row_blk(DH), _full_blk(1, 2 * DH),
              _full_blk(2 * DH, 2 * DH), _full_blk(1, 2 * DH)],
    out_specs=pl.BlockSpec((RB, 2 * DH), lambda i: (i, 0)),
    out_shape=jax.ShapeDtypeStruct((N, 2 * DH), f32),
)


def kernel(x, edge_index, W1, b1, W2, b2, W3, b3, Wout, bout):
    src = edge_index[0].astype(jnp.int32)
    dst = edge_index[1].astype(jnp.int32)
    pad = EPAD - E
    src_p = jnp.concatenate([src, jnp.zeros((pad,), jnp.int32)])
    dst_p = jnp.concatenate([dst, jnp.full((pad,), N, jnp.int32)])
    dst_t = dst_p.reshape(NS, NCHUNK, CHUNK)
    pk_t = (jnp.left_shift(dst_p, 14) | src_p).reshape(NS, NCHUNK, CHUNK)

    p0, p1 = _deg_kernel(dst_t)

    b1r = b1.reshape(1, 2 * DH)
    b2r = b2.reshape(1, 2 * DH)
    b3r = b3.reshape(1, 2 * DH)
    boutr = bout.reshape(1, 2 * DH)

    g0, g1 = _tc_in(x, W1, p0, p1)
    s0, s1 = _agg_kernel(g0, g1, pk_t)
    g0, g1 = _tc_mid(s0, s1, g0, g1, p0, p1, b1r, W2)
    s0, s1 = _agg_kernel(g0, g1, pk_t)
    g0, g1 = _tc_mid(s0, s1, g0, g1, p0, p1, b2r, W3)
    s0, s1 = _agg_kernel(g0, g1, pk_t)
    return _tc_out(s0, s1, g0, g1, p0, p1, b3r, Wout, boutr)
